# Initial kernel scaffold; baseline (speedup 1.0000x reference)
import jax
import jax.numpy as jnp
from jax.experimental import pallas as pl


def kernel(x_node, x_edge, edge_index, W1, b1, W2, b2, V1, c1, V2, c2):
    raise NotImplementedError("write your pallas kernel here")


# speedup vs baseline: 3.5291x; 3.5291x over previous
import functools

import jax
import jax.numpy as jnp
from jax import lax
from jax.experimental import pallas as pl
from jax.experimental.pallas import tpu as pltpu
from jax.experimental.pallas import tpu_sc as plsc

N_NODES = 100000
D = 32

_NC = 2
_NS = 16
_LANES = 16

_GW = 80

_HALF = N_NODES // _NC
_ACC_ROWS = 50176
_ZROWS = _ACC_ROWS // _NS
_DUMP_ROWS = _HALF // _NS

_BE = 3200


def _sc_gather(x_node, src2d, dst2d):
    n_win = src2d.shape[0]
    e = n_win * _GW
    mesh = plsc.VectorSubcoreMesh(core_axis_name="c", subcore_axis_name="s")

    @functools.partial(
        pl.kernel,
        out_type=(jax.ShapeDtypeStruct((e, D), jnp.float32),
                  jax.ShapeDtypeStruct((e, D), jnp.float32)),
        mesh=mesh,
        compiler_params=pltpu.CompilerParams(use_tc_tiling_on_sc=False),
    )
    def gk(xn_hbm, src_hbm, dst_hbm, xs_hbm, xd_hbm):
        def body(is_vmem, id_vmem, xs_vmem, xd_vmem):
            pltpu.sync_copy(xn_hbm.at[is_vmem.at[0]], xs_vmem)
            pltpu.sync_copy(xn_hbm.at[id_vmem.at[0]], xd_vmem)

        pltpu.emit_pipeline(
            body,
            grid=(n_win,),
            in_specs=[pl.BlockSpec((1, _GW), lambda i: (i, 0)),
                      pl.BlockSpec((1, _GW), lambda i: (i, 0))],
            out_specs=[pl.BlockSpec((_GW, D), lambda i: (i, 0)),
                       pl.BlockSpec((_GW, D), lambda i: (i, 0))],
            core_axis_name=("c", "s"),
            dimension_semantics=(pltpu.PARALLEL,),
        )(src_hbm, dst_hbm, xs_hbm, xd_hbm)

    return gk(x_node, src2d, dst2d)


def _tc_mlp(xd, xs, xe, v1a, v1b, v1c, c1, v2, c2, w1a, w1b, b1, w2, b2):
    e = xd.shape[0]
    grid = e // _BE

    def body(xd_ref, xs_ref, xe_ref, v1a_ref, v1b_ref, v1c_ref, c1_ref,
             v2_ref, c2_ref, w1a_ref, w1b_ref, b1_ref, w2_ref, b2_ref,
             em_ref, msg_ref):
        xd_b = xd_ref[...]
        xs_b = xs_ref[...]
        xe_b = xe_ref[...]
        dot = functools.partial(jnp.dot, preferred_element_type=jnp.float32)
        h = (dot(xd_b, v1a_ref[...]) + dot(xs_b, v1b_ref[...])
             + dot(xe_b, v1c_ref[...]) + c1_ref[...])
        h = jnp.maximum(h, 0.0)
        em_b = dot(h, v2_ref[...]) + c2_ref[...]
        em_ref[...] = em_b
        h2 = dot(xd_b, w1a_ref[...]) + dot(em_b, w1b_ref[...]) + b1_ref[...]
        h2 = jnp.maximum(h2, 0.0)
        msg_ref[...] = dot(h2, w2_ref[...]) + b2_ref[...]

    full = lambda shape: pl.BlockSpec(shape, lambda i: tuple(0 for _ in shape))
    row = lambda width: pl.BlockSpec((_BE, width), lambda i: (i, 0))
    return pl.pallas_call(
        body,
        grid=(grid,),
        in_specs=[row(D), row(D), row(6),
                  full(v1a.shape), full(v1b.shape), full(v1c.shape),
                  full(c1.shape), full(v2.shape), full(c2.shape),
                  full(w1a.shape), full(w1b.shape), full(b1.shape),
                  full(w2.shape), full(b2.shape)],
        out_specs=[row(6), row(D)],
        out_shape=[jax.ShapeDtypeStruct((e, 6), jnp.float32),
                   jax.ShapeDtypeStruct((e, D), jnp.float32)],
    )(xd, xs, xe, v1a, v1b, v1c, c1, v2, c2, w1a, w1b, b1, w2, b2)


def _sc_scatter(msg, dst2d, zeros):
    n_win = dst2d.shape[0]
    rows_per_tile = n_win // _NS
    chunk = 10
    n_outer = rows_per_tile // chunk
    mesh = plsc.VectorSubcoreMesh(core_axis_name="c", subcore_axis_name="s")

    @functools.partial(
        pl.kernel,
        out_type=jax.ShapeDtypeStruct((N_NODES, D), jnp.float32),
        mesh=mesh,
        compiler_params=pltpu.CompilerParams(use_tc_tiling_on_sc=False),
        scratch_types=[
            pltpu.VMEM_SHARED((_ACC_ROWS, D), jnp.float32),
            pltpu.VMEM((chunk * _GW, D), jnp.float32),
            pltpu.VMEM((chunk, _GW), jnp.int32),
            pltpu.SemaphoreType.DMA,
            pltpu.SemaphoreType.DMA,
        ],
    )
    def sk(msg_hbm, dst_hbm, z_hbm, nm_hbm, acc, mbuf, ibuf, lsem, ssem):
        c = lax.axis_index("c")
        s = lax.axis_index("s")
        base = c * _HALF
        pltpu.sync_copy(z_hbm, acc.at[pl.ds(s * _ZROWS, _ZROWS), :])
        plsc.subcore_barrier()

        row_lo = s * rows_per_tile

        @pl.loop(0, n_outer)
        def _(g):
            r0 = row_lo + g * chunk
            cp_i = pltpu.async_copy(dst_hbm.at[pl.ds(r0, chunk), :], ibuf, lsem)
            cp_m = pltpu.async_copy(
                msg_hbm.at[pl.ds(r0 * _GW, chunk * _GW), :], mbuf, lsem)
            cp_i.wait()
            cp_m.wait()
            for j in range(chunk):
                for i in range(_GW // _LANES):
                    v = ibuf[j, pl.ds(i * _LANES, _LANES)]
                    ok = (v >= base) & (v < base + _HALF)
                    ibuf[j, pl.ds(i * _LANES, _LANES)] = jnp.where(
                        ok, v - base, _HALF)
            cps = [
                pltpu.async_copy(mbuf.at[pl.ds(j * _GW, _GW), :],
                                 acc.at[ibuf.at[j]], ssem, add=True)
                for j in range(chunk)
            ]
            for cp in cps:
                cp.wait()

        plsc.subcore_barrier()
        pltpu.sync_copy(
            acc.at[pl.ds(s * _DUMP_ROWS, _DUMP_ROWS), :],
            nm_hbm.at[pl.ds(base + s * _DUMP_ROWS, _DUMP_ROWS), :])

    return sk(msg, dst2d, zeros)


def kernel(x_node, x_edge, edge_index, W1, b1, W2, b2, V1, c1, V2, c2):
    src2d = edge_index[0].reshape(-1, _GW)
    dst2d = edge_index[1].reshape(-1, _GW)
    xs, xd = _sc_gather(x_node, src2d, dst2d)
    em, msg = _tc_mlp(
        xd, xs, x_edge,
        V1[0:D], V1[D:2 * D], V1[2 * D:], c1.reshape(1, -1),
        V2, c2.reshape(1, -1),
        W1[0:D], W1[D:], b1.reshape(1, -1),
        W2, b2.reshape(1, -1))
    zeros = jnp.zeros((_ZROWS, D), jnp.float32)
    nm = _sc_scatter(msg, dst2d, zeros)
    return (nm, em)
